# baseline (speedup 1.0000x reference)
import functools

import jax
import jax.numpy as jnp
from jax import lax
from jax.experimental import pallas as pl
from jax.experimental.pallas import tpu as pltpu
from jax.experimental.pallas import tpu_sc as plsc

B = 16384
L = 50
LPAD = 56
D = 64
NCLS = 100
NBUF = 4


def _tc_pad128(x):
  bm = 2048

  def body(x_ref, o_ref):
    xb = x_ref[...]
    o_ref[...] = jnp.concatenate(
        [xb, xb[:, : LPAD - L],
         jnp.zeros((bm, 128 - LPAD), jnp.int32)], axis=1)

  return pl.pallas_call(
      body,
      grid=(B // bm,),
      in_specs=[pl.BlockSpec((bm, L), lambda i: (i, 0))],
      out_specs=pl.BlockSpec((bm, 128), lambda i: (i, 0)),
      out_shape=jax.ShapeDtypeStruct((B, 128), jnp.int32),
  )(x)


def _sc_pool(xp, table):
  info = plsc.get_sparse_core_info()
  nc, ns = info.num_cores, info.num_subcores
  nw = nc * ns
  spw = B // nw
  mesh = plsc.VectorSubcoreMesh(core_axis_name="c", subcore_axis_name="s")

  @functools.partial(
      pl.kernel,
      out_type=jax.ShapeDtypeStruct((B, D), jnp.float32),
      mesh=mesh,
      scratch_types=[
          pltpu.VMEM((spw, 128), jnp.int32),
          pltpu.VMEM((NBUF, LPAD, D), jnp.float32),
          pltpu.VMEM((spw, D), jnp.float32),
          pltpu.SemaphoreType.DMA((NBUF,)),
      ],
      compiler_params=pltpu.CompilerParams(use_tc_tiling_on_sc=False),
  )
  def k(x_hbm, table_hbm, out_hbm, idx_a, rows_v, pooled_v, sems):
    wid = lax.axis_index("s") * nc + lax.axis_index("c")
    base = wid * spw
    pltpu.sync_copy(x_hbm.at[pl.ds(base, spw)], idx_a)

    def start(s, b):
      pltpu.async_copy(
          table_hbm.at[idx_a.at[s, pl.ds(0, LPAD)]], rows_v.at[b],
          sems.at[b])

    def wait(b):
      pltpu.make_async_copy(
          table_hbm.at[pl.ds(0, LPAD)], rows_v.at[b], sems.at[b]).wait()

    def accum(s, b):
      accs = [rows_v[b, 0, pl.ds(16 * q, 16)] for q in range(D // 16)]
      for j in range(1, L):
        for q in range(D // 16):
          accs[q] = accs[q] + rows_v[b, j, pl.ds(16 * q, 16)]
      for q in range(D // 16):
        pooled_v[s, pl.ds(16 * q, 16)] = accs[q]

    for b in range(NBUF):
      start(b, b)

    def body(t, _):
      for b in range(NBUF):
        s = NBUF * t + b
        wait(b)
        accum(s, b)

        @pl.when(s + NBUF < spw)
        def _():
          start(s + NBUF, b)

      return 0

    lax.fori_loop(0, spw // NBUF, body, 0)
    pltpu.sync_copy(pooled_v, out_hbm.at[pl.ds(base, spw)])

  return k(xp, table)


def _tc_head(pooled, wt, b2):
  bm = 1024

  def body(p_ref, w_ref, b_ref, o_ref):
    acc = jnp.dot(p_ref[...], w_ref[...], preferred_element_type=jnp.float32)
    o_ref[...] = acc * (1.0 / L) + b_ref[...]

  return pl.pallas_call(
      body,
      grid=(B // bm,),
      in_specs=[
          pl.BlockSpec((bm, D), lambda i: (i, 0)),
          pl.BlockSpec((D, NCLS), lambda i: (0, 0)),
          pl.BlockSpec((1, NCLS), lambda i: (0, 0)),
      ],
      out_specs=pl.BlockSpec((bm, NCLS), lambda i: (i, 0)),
      out_shape=jax.ShapeDtypeStruct((B, NCLS), jnp.float32),
  )(pooled, wt, b2)


def kernel(x, table, W, b):
  xp = _tc_pad128(x.astype(jnp.int32))
  pooled = _sc_pool(xp, table)
  return _tc_head(pooled, W.T, b.reshape(1, NCLS))


# speedup vs baseline: 1.0066x; 1.0066x over previous
import functools

import jax
import jax.numpy as jnp
from jax import lax
from jax.experimental import pallas as pl
from jax.experimental.pallas import tpu as pltpu
from jax.experimental.pallas import tpu_sc as plsc

B = 16384
L = 50
D = 64
NCLS = 100
NBUF = 4


def _sc_pool(x, table):
  info = plsc.get_sparse_core_info()
  nc, ns = info.num_cores, info.num_subcores
  nw = nc * ns
  spw = B // nw
  mesh = plsc.VectorSubcoreMesh(core_axis_name="c", subcore_axis_name="s")

  @functools.partial(
      pl.kernel,
      out_type=jax.ShapeDtypeStruct((B, D), jnp.float32),
      mesh=mesh,
      scratch_types=[
          pltpu.VMEM((spw, L), jnp.int32),
          pltpu.VMEM((NBUF, L, D), jnp.float32),
          pltpu.VMEM((spw, D), jnp.float32),
          pltpu.SemaphoreType.DMA((NBUF,)),
      ],
      compiler_params=pltpu.CompilerParams(use_tc_tiling_on_sc=False),
  )
  def k(x_hbm, table_hbm, out_hbm, idx_a, rows_v, pooled_v, sems):
    wid = lax.axis_index("s") * nc + lax.axis_index("c")
    base = wid * spw
    pltpu.sync_copy(x_hbm.at[pl.ds(base, spw)], idx_a)

    def start(s, b):
      pltpu.async_copy(
          table_hbm.at[idx_a.at[s]], rows_v.at[b], sems.at[b])

    def wait(b):
      pltpu.make_async_copy(
          table_hbm.at[pl.ds(0, L)], rows_v.at[b], sems.at[b]).wait()

    def accum(s, b):
      accs = [rows_v[b, 0, pl.ds(16 * q, 16)] for q in range(D // 16)]
      for j in range(1, L):
        for q in range(D // 16):
          accs[q] = accs[q] + rows_v[b, j, pl.ds(16 * q, 16)]
      for q in range(D // 16):
        pooled_v[s, pl.ds(16 * q, 16)] = accs[q]

    for b in range(NBUF):
      start(b, b)

    def body(t, _):
      for b in range(NBUF):
        s = NBUF * t + b
        wait(b)
        accum(s, b)

        @pl.when(s + NBUF < spw)
        def _():
          start(s + NBUF, b)

      return 0

    lax.fori_loop(0, spw // NBUF, body, 0)
    pltpu.sync_copy(pooled_v, out_hbm.at[pl.ds(base, spw)])

  return k(x, table)


def _tc_head(pooled, wt, b2):
  bm = 1024

  def body(p_ref, w_ref, b_ref, o_ref):
    acc = jnp.dot(p_ref[...], w_ref[...], preferred_element_type=jnp.float32)
    o_ref[...] = acc * (1.0 / L) + b_ref[...]

  return pl.pallas_call(
      body,
      grid=(B // bm,),
      in_specs=[
          pl.BlockSpec((bm, D), lambda i: (i, 0)),
          pl.BlockSpec((D, NCLS), lambda i: (0, 0)),
          pl.BlockSpec((1, NCLS), lambda i: (0, 0)),
      ],
      out_specs=pl.BlockSpec((bm, NCLS), lambda i: (i, 0)),
      out_shape=jax.ShapeDtypeStruct((B, NCLS), jnp.float32),
  )(pooled, wt, b2)


def kernel(x, table, W, b):
  pooled = _sc_pool(x.astype(jnp.int32), table)
  return _tc_head(pooled, W.T, b.reshape(1, NCLS))
